# baseline (speedup 1.0000x reference)
import functools

import jax
import jax.numpy as jnp
from jax import lax
from jax.experimental import pallas as pl
from jax.experimental.pallas import tpu as pltpu
from jax.experimental.pallas import tpu_sc as plsc

N_NODES = 10000
N_EDGES = 320000
D_IN = 128
DH = 64
PW = 16

NC = 2
NS = 16
CHUNK = 128
NPAD = 16
NROW = N_NODES + NPAD
RPT = N_NODES // NS
EPT1 = N_EDGES // NS
NCH1 = 160
EPT2 = N_EDGES // (NC * NS)
NCH2 = 80

_SC_PARAMS = pltpu.CompilerParams(use_tc_tiling_on_sc=False)


def _sc_pass1(xs, src_r, dst_r, zrow, z16):
  mesh = plsc.VectorSubcoreMesh(core_axis_name="c", subcore_axis_name="s")
  out_type = [
      jax.ShapeDtypeStruct((NC, N_NODES, DH), jnp.float32),
      jax.ShapeDtypeStruct((NC, N_NODES, PW), jnp.float32),
  ]
  scratch = [
      pltpu.VMEM_SHARED((NROW, DH), jnp.float32),
      pltpu.VMEM_SHARED((NROW, PW), jnp.float32),
      pltpu.VMEM((NCH1, CHUNK), jnp.int32),
      pltpu.VMEM((NCH1, CHUNK), jnp.int32),
      pltpu.VMEM((4, CHUNK, DH), jnp.float32),
      pltpu.VMEM((CHUNK, PW), jnp.float32),
      pltpu.SemaphoreType.DMA((4,)),
      pltpu.SemaphoreType.DMA((4,)),
      pltpu.SemaphoreType.DMA,
  ]

  @functools.partial(pl.kernel, out_type=out_type, mesh=mesh,
                     scratch_types=scratch, compiler_params=_SC_PARAMS)
  def body(x_h, src_h, dst_h, zrow_h, z16_h, out_h, outc_h,
           acc, accc, src_v, dst_v, rows, ones_v, gsem, ssem, csem):
    c = lax.axis_index("c")
    s = lax.axis_index("s")

    pltpu.sync_copy(src_h.at[s], src_v)
    pltpu.sync_copy(dst_h.at[s], dst_v)
    pltpu.sync_copy(zrow_h, acc.at[pl.ds(s * RPT, RPT)])
    pltpu.sync_copy(z16_h, accc.at[pl.ds(s * RPT, RPT)])
    for j in range(CHUNK):
      ones_v[j, :] = jnp.ones((PW,), jnp.float32)
    plsc.subcore_barrier()

    table = x_h.at[c]
    pltpu.async_copy(table.at[src_v.at[0]], rows.at[0], gsem.at[0])

    def group(g, carry):
      for b in range(4):
        i = g * 4 + b
        b1 = (b + 1) % 4
        pltpu.make_async_copy(table.at[src_v.at[i]], rows.at[b],
                              gsem.at[b]).wait()
        pltpu.async_copy(rows.at[b], acc.at[dst_v.at[i]], ssem.at[b],
                         add=True)

        @pl.when(i // (NCH1 // NC) == c)
        def _():
          pltpu.async_copy(ones_v, accc.at[dst_v.at[i]], csem, add=True)

        @pl.when(i >= 3)
        def _():
          pltpu.make_async_copy(rows.at[b1], acc.at[dst_v.at[0]],
                                ssem.at[b1]).wait()

        @pl.when(i + 1 < NCH1)
        def _():
          pltpu.async_copy(table.at[src_v.at[i + 1]], rows.at[b1],
                           gsem.at[b1])

      return carry

    lax.fori_loop(0, NCH1 // 4, group, 0)
    for b in (1, 2, 3):
      pltpu.make_async_copy(rows.at[b], acc.at[dst_v.at[0]],
                            ssem.at[b]).wait()

    def cdrain(k, carry):
      pltpu.make_async_copy(ones_v, accc.at[dst_v.at[0]], csem).wait()
      return carry

    lax.fori_loop(0, NCH1 // NC, cdrain, 0)
    plsc.subcore_barrier()

    pltpu.sync_copy(acc.at[pl.ds(s * RPT, RPT)],
                    out_h.at[c, pl.ds(s * RPT, RPT)])
    pltpu.sync_copy(accc.at[pl.ds(s * RPT, RPT)],
                    outc_h.at[c, pl.ds(s * RPT, RPT)])

  return body(xs, src_r, dst_r, zrow, z16)


def _sc_pass2(p, src_r, dst_r, z16):
  mesh = plsc.VectorSubcoreMesh(core_axis_name="c", subcore_axis_name="s")
  out_type = [jax.ShapeDtypeStruct((NC, N_NODES, PW), jnp.float32)]
  scratch = [
      pltpu.VMEM_SHARED((NROW, PW), jnp.float32),
      pltpu.VMEM((NCH2, CHUNK), jnp.int32),
      pltpu.VMEM((NCH2, CHUNK), jnp.int32),
      pltpu.VMEM((4, CHUNK, PW), jnp.float32),
      pltpu.SemaphoreType.DMA((4,)),
      pltpu.SemaphoreType.DMA((4,)),
  ]

  @functools.partial(pl.kernel, out_type=out_type, mesh=mesh,
                     scratch_types=scratch, compiler_params=_SC_PARAMS)
  def body(p_h, src_h, dst_h, z16_h, out_h, acc, src_v, dst_v, rows,
           gsem, ssem):
    c = lax.axis_index("c")
    s = lax.axis_index("s")

    pltpu.sync_copy(src_h.at[s, pl.ds(c * NCH2, NCH2)], src_v)
    pltpu.sync_copy(dst_h.at[s, pl.ds(c * NCH2, NCH2)], dst_v)
    pltpu.sync_copy(z16_h, acc.at[pl.ds(s * RPT, RPT)])
    plsc.subcore_barrier()

    pltpu.async_copy(p_h.at[src_v.at[0]], rows.at[0], gsem.at[0])

    def group(g, carry):
      for b in range(4):
        i = g * 4 + b
        b1 = (b + 1) % 4
        pltpu.make_async_copy(p_h.at[src_v.at[i]], rows.at[b],
                              gsem.at[b]).wait()
        pltpu.async_copy(rows.at[b], acc.at[dst_v.at[i]], ssem.at[b],
                         add=True)

        @pl.when(i >= 3)
        def _():
          pltpu.make_async_copy(rows.at[b1], acc.at[dst_v.at[0]],
                                ssem.at[b1]).wait()

        @pl.when(i + 1 < NCH2)
        def _():
          pltpu.async_copy(p_h.at[src_v.at[i + 1]], rows.at[b1],
                           gsem.at[b1])

      return carry

    lax.fori_loop(0, NCH2 // 4, group, 0)
    for b in (1, 2, 3):
      pltpu.make_async_copy(rows.at[b], acc.at[dst_v.at[0]],
                            ssem.at[b]).wait()
    plsc.subcore_barrier()

    pltpu.sync_copy(acc.at[pl.ds(s * RPT, RPT)],
                    out_h.at[c, pl.ds(s * RPT, RPT)])

  return body(p, src_r, dst_r, z16)


def _tc_mid(part1, cntp, x, w1la, w1lb, b1r, W1r, w2lp, w2rp, b2p):
  BR = 1000
  G = N_NODES // BR

  def body(p1_ref, cp_ref, x_ref, w1la_ref, w1lb_ref, b1_ref, w1r_ref,
           w2l_ref, w2r_ref, b2_ref, p_ref, z_ref, inv_ref):
    cnt16 = cp_ref[0] + cp_ref[1]
    inv16 = 1.0 / jnp.maximum(cnt16, 1.0)
    inv = inv16[:, 0:1]
    h = ((p1_ref[0] * inv) @ w1la_ref[...]
         + (p1_ref[1] * inv) @ w1lb_ref[...]
         + x_ref[...] @ w1r_ref[...] + b1_ref[...])
    h = jnp.maximum(h, 0.0)
    zpad = jnp.zeros((BR, PW - 3), jnp.float32)
    p_ref[...] = jnp.concatenate([h @ w2l_ref[...], zpad], axis=1)
    z_ref[...] = jnp.concatenate([h @ w2r_ref[...] + b2_ref[...], zpad],
                                 axis=1)
    inv_ref[...] = inv16[:, 0:8]

  return pl.pallas_call(
      body,
      grid=(G,),
      in_specs=[
          pl.BlockSpec((NC, BR, DH), lambda i: (0, i, 0)),
          pl.BlockSpec((NC, BR, PW), lambda i: (0, i, 0)),
          pl.BlockSpec((BR, D_IN), lambda i: (i, 0)),
          pl.BlockSpec((DH, D_IN), lambda i: (0, 0)),
          pl.BlockSpec((DH, D_IN), lambda i: (0, 0)),
          pl.BlockSpec((1, D_IN), lambda i: (0, 0)),
          pl.BlockSpec((D_IN, D_IN), lambda i: (0, 0)),
          pl.BlockSpec((D_IN, 3), lambda i: (0, 0)),
          pl.BlockSpec((D_IN, 3), lambda i: (0, 0)),
          pl.BlockSpec((1, 3), lambda i: (0, 0)),
      ],
      out_specs=[
          pl.BlockSpec((BR, PW), lambda i: (i, 0)),
          pl.BlockSpec((BR, PW), lambda i: (i, 0)),
          pl.BlockSpec((BR, 8), lambda i: (i, 0)),
      ],
      out_shape=[
          jax.ShapeDtypeStruct((N_NODES, PW), jnp.float32),
          jax.ShapeDtypeStruct((N_NODES, PW), jnp.float32),
          jax.ShapeDtypeStruct((N_NODES, 8), jnp.float32),
      ],
  )(part1, cntp, x, w1la, w1lb, b1r, W1r, w2lp, w2rp, b2p)


def _tc_final(part2, z, inv):
  BR = 1000
  G = N_NODES // BR

  def body(p2_ref, z_ref, inv_ref, o_ref):
    agg = p2_ref[0] + p2_ref[1]
    o_ref[...] = agg * inv_ref[:, 0:1] + z_ref[...]

  return pl.pallas_call(
      body,
      grid=(G,),
      in_specs=[
          pl.BlockSpec((NC, BR, PW), lambda i: (0, i, 0)),
          pl.BlockSpec((BR, PW), lambda i: (i, 0)),
          pl.BlockSpec((BR, 8), lambda i: (i, 0)),
      ],
      out_specs=pl.BlockSpec((BR, PW), lambda i: (i, 0)),
      out_shape=jax.ShapeDtypeStruct((N_NODES, PW), jnp.float32),
  )(part2, z, inv)


def _impl(x, edge_index, W1l, b1, W1r, W2l, b2, W2r):
  ei = edge_index.astype(jnp.int32)
  pad1 = NCH1 * CHUNK - EPT1
  dmy1 = jnp.broadcast_to(N_NODES + jnp.arange(pad1, dtype=jnp.int32) % NPAD,
                          (NS, pad1))
  src1 = jnp.concatenate(
      [ei[0].reshape(NS, EPT1), jnp.zeros((NS, pad1), jnp.int32)],
      axis=1).reshape(NS, NCH1, CHUNK)
  dst1 = jnp.concatenate(
      [ei[1].reshape(NS, EPT1), dmy1], axis=1).reshape(NS, NCH1, CHUNK)
  z64 = jnp.zeros((RPT, DH), jnp.float32)
  z16 = jnp.zeros((RPT, PW), jnp.float32)
  w1la = W1l[:DH]
  w1lb = W1l[DH:]
  b1r = b1.reshape(1, D_IN)
  b2r = b2.reshape(1, 3)

  xs = jnp.stack([x[:, :DH], x[:, DH:]])
  part1, cntp = _sc_pass1(xs, src1, dst1, z64, z16)
  p, zz, inv = _tc_mid(part1, cntp, x, w1la, w1lb, b1r, W1r, W2l, W2r, b2r)
  (part2,) = _sc_pass2(p, src1, dst1, z16)
  out16 = _tc_final(part2, zz, inv)
  return out16[:, :3]


kernel = jax.jit(_impl)


# speedup vs baseline: 1.2214x; 1.2214x over previous
import functools

import jax
import jax.numpy as jnp
from jax import lax
from jax.experimental import pallas as pl
from jax.experimental.pallas import tpu as pltpu
from jax.experimental.pallas import tpu_sc as plsc

N_NODES = 10000
N_EDGES = 320000
D_IN = 128
DH = 64
PW = 16

NC = 2
NS = 16
CHUNK = 128
NPAD = 16
NROW = N_NODES + NPAD
RPT = N_NODES // NS
EPT1 = N_EDGES // NS
NCH1 = 160
EPT2 = N_EDGES // (NC * NS)
NCH2 = 80

_SC_PARAMS = pltpu.CompilerParams(use_tc_tiling_on_sc=False)


def _sc_pass1(xs, src_r, dst_r, zrow, z16):
  mesh = plsc.VectorSubcoreMesh(core_axis_name="c", subcore_axis_name="s")
  out_type = [
      jax.ShapeDtypeStruct((NC, N_NODES, DH), jnp.float32),
      jax.ShapeDtypeStruct((NC, N_NODES, PW), jnp.float32),
  ]
  scratch = [
      pltpu.VMEM_SHARED((NROW, DH), jnp.float32),
      pltpu.VMEM_SHARED((NROW, PW), jnp.float32),
      pltpu.VMEM((NCH1, CHUNK), jnp.int32),
      pltpu.VMEM((NCH1, CHUNK), jnp.int32),
      pltpu.VMEM((4, CHUNK, DH), jnp.float32),
      pltpu.VMEM((CHUNK, PW), jnp.float32),
      pltpu.SemaphoreType.DMA((4,)),
      pltpu.SemaphoreType.DMA((4,)),
      pltpu.SemaphoreType.DMA,
  ]

  @functools.partial(pl.kernel, out_type=out_type, mesh=mesh,
                     scratch_types=scratch, compiler_params=_SC_PARAMS)
  def body(x_h, src_h, dst_h, zrow_h, z16_h, out_h, outc_h,
           acc, accc, src_v, dst_v, rows, ones_v, gsem, ssem, csem):
    c = lax.axis_index("c")
    s = lax.axis_index("s")

    pltpu.sync_copy(src_h.at[s], src_v)
    pltpu.sync_copy(dst_h.at[s], dst_v)
    pltpu.sync_copy(zrow_h, acc.at[pl.ds(s * RPT, RPT)])
    pltpu.sync_copy(z16_h, accc.at[pl.ds(s * RPT, RPT)])
    for j in range(CHUNK):
      ones_v[j, :] = jnp.ones((PW,), jnp.float32)
    plsc.subcore_barrier()

    table = x_h.at[c]
    for b in range(3):
      pltpu.async_copy(table.at[src_v.at[b]], rows.at[b], gsem.at[b])

    def group(g, carry):
      for b in range(4):
        i = g * 4 + b
        b3 = (b + 3) % 4
        pltpu.make_async_copy(table.at[src_v.at[i]], rows.at[b],
                              gsem.at[b]).wait()
        pltpu.async_copy(rows.at[b], acc.at[dst_v.at[i]], ssem.at[b],
                         add=True)

        @pl.when(i // (NCH1 // NC) == c)
        def _():
          pltpu.async_copy(ones_v, accc.at[dst_v.at[i]], csem, add=True)

        @pl.when(i >= 1)
        def _():
          pltpu.make_async_copy(rows.at[b3], acc.at[dst_v.at[0]],
                                ssem.at[b3]).wait()

        @pl.when(i + 3 < NCH1)
        def _():
          pltpu.async_copy(table.at[src_v.at[i + 3]], rows.at[b3],
                           gsem.at[b3])

      return carry

    lax.fori_loop(0, NCH1 // 4, group, 0)
    for b in (3,):
      pltpu.make_async_copy(rows.at[b], acc.at[dst_v.at[0]],
                            ssem.at[b]).wait()

    def cdrain(k, carry):
      pltpu.make_async_copy(ones_v, accc.at[dst_v.at[0]], csem).wait()
      return carry

    lax.fori_loop(0, NCH1 // NC, cdrain, 0)
    plsc.subcore_barrier()

    pltpu.sync_copy(acc.at[pl.ds(s * RPT, RPT)],
                    out_h.at[c, pl.ds(s * RPT, RPT)])
    pltpu.sync_copy(accc.at[pl.ds(s * RPT, RPT)],
                    outc_h.at[c, pl.ds(s * RPT, RPT)])

  return body(xs, src_r, dst_r, zrow, z16)


def _sc_pass2(p, src_r, dst_r, z16):
  mesh = plsc.VectorSubcoreMesh(core_axis_name="c", subcore_axis_name="s")
  out_type = [jax.ShapeDtypeStruct((NC, N_NODES, PW), jnp.float32)]
  scratch = [
      pltpu.VMEM_SHARED((NROW, PW), jnp.float32),
      pltpu.VMEM((NCH2, CHUNK), jnp.int32),
      pltpu.VMEM((NCH2, CHUNK), jnp.int32),
      pltpu.VMEM((4, CHUNK, PW), jnp.float32),
      pltpu.SemaphoreType.DMA((4,)),
      pltpu.SemaphoreType.DMA((4,)),
  ]

  @functools.partial(pl.kernel, out_type=out_type, mesh=mesh,
                     scratch_types=scratch, compiler_params=_SC_PARAMS)
  def body(p_h, src_h, dst_h, z16_h, out_h, acc, src_v, dst_v, rows,
           gsem, ssem):
    c = lax.axis_index("c")
    s = lax.axis_index("s")

    pltpu.sync_copy(src_h.at[s, pl.ds(c * NCH2, NCH2)], src_v)
    pltpu.sync_copy(dst_h.at[s, pl.ds(c * NCH2, NCH2)], dst_v)
    pltpu.sync_copy(z16_h, acc.at[pl.ds(s * RPT, RPT)])
    plsc.subcore_barrier()

    for b in range(3):
      pltpu.async_copy(p_h.at[src_v.at[b]], rows.at[b], gsem.at[b])

    def group(g, carry):
      for b in range(4):
        i = g * 4 + b
        b3 = (b + 3) % 4
        pltpu.make_async_copy(p_h.at[src_v.at[i]], rows.at[b],
                              gsem.at[b]).wait()
        pltpu.async_copy(rows.at[b], acc.at[dst_v.at[i]], ssem.at[b],
                         add=True)

        @pl.when(i >= 1)
        def _():
          pltpu.make_async_copy(rows.at[b3], acc.at[dst_v.at[0]],
                                ssem.at[b3]).wait()

        @pl.when(i + 3 < NCH2)
        def _():
          pltpu.async_copy(p_h.at[src_v.at[i + 3]], rows.at[b3],
                           gsem.at[b3])

      return carry

    lax.fori_loop(0, NCH2 // 4, group, 0)
    for b in (3,):
      pltpu.make_async_copy(rows.at[b], acc.at[dst_v.at[0]],
                            ssem.at[b]).wait()
    plsc.subcore_barrier()

    pltpu.sync_copy(acc.at[pl.ds(s * RPT, RPT)],
                    out_h.at[c, pl.ds(s * RPT, RPT)])

  return body(p, src_r, dst_r, z16)


def _tc_mid(part1, cntp, x, w1la, w1lb, b1r, W1r, w2lp, w2rp, b2p):
  BR = 1000
  G = N_NODES // BR

  def body(p1_ref, cp_ref, x_ref, w1la_ref, w1lb_ref, b1_ref, w1r_ref,
           w2l_ref, w2r_ref, b2_ref, p_ref, z_ref, inv_ref):
    cnt16 = cp_ref[0] + cp_ref[1]
    inv16 = 1.0 / jnp.maximum(cnt16, 1.0)
    inv = inv16[:, 0:1]
    h = ((p1_ref[0] * inv) @ w1la_ref[...]
         + (p1_ref[1] * inv) @ w1lb_ref[...]
         + x_ref[...] @ w1r_ref[...] + b1_ref[...])
    h = jnp.maximum(h, 0.0)
    zpad = jnp.zeros((BR, PW - 3), jnp.float32)
    p_ref[...] = jnp.concatenate([h @ w2l_ref[...], zpad], axis=1)
    z_ref[...] = jnp.concatenate([h @ w2r_ref[...] + b2_ref[...], zpad],
                                 axis=1)
    inv_ref[...] = inv16[:, 0:8]

  return pl.pallas_call(
      body,
      grid=(G,),
      in_specs=[
          pl.BlockSpec((NC, BR, DH), lambda i: (0, i, 0)),
          pl.BlockSpec((NC, BR, PW), lambda i: (0, i, 0)),
          pl.BlockSpec((BR, D_IN), lambda i: (i, 0)),
          pl.BlockSpec((DH, D_IN), lambda i: (0, 0)),
          pl.BlockSpec((DH, D_IN), lambda i: (0, 0)),
          pl.BlockSpec((1, D_IN), lambda i: (0, 0)),
          pl.BlockSpec((D_IN, D_IN), lambda i: (0, 0)),
          pl.BlockSpec((D_IN, 3), lambda i: (0, 0)),
          pl.BlockSpec((D_IN, 3), lambda i: (0, 0)),
          pl.BlockSpec((1, 3), lambda i: (0, 0)),
      ],
      out_specs=[
          pl.BlockSpec((BR, PW), lambda i: (i, 0)),
          pl.BlockSpec((BR, PW), lambda i: (i, 0)),
          pl.BlockSpec((BR, 8), lambda i: (i, 0)),
      ],
      out_shape=[
          jax.ShapeDtypeStruct((N_NODES, PW), jnp.float32),
          jax.ShapeDtypeStruct((N_NODES, PW), jnp.float32),
          jax.ShapeDtypeStruct((N_NODES, 8), jnp.float32),
      ],
  )(part1, cntp, x, w1la, w1lb, b1r, W1r, w2lp, w2rp, b2p)


def _tc_final(part2, z, inv):
  BR = 1000
  G = N_NODES // BR

  def body(p2_ref, z_ref, inv_ref, o_ref):
    agg = p2_ref[0] + p2_ref[1]
    o_ref[...] = agg * inv_ref[:, 0:1] + z_ref[...]

  return pl.pallas_call(
      body,
      grid=(G,),
      in_specs=[
          pl.BlockSpec((NC, BR, PW), lambda i: (0, i, 0)),
          pl.BlockSpec((BR, PW), lambda i: (i, 0)),
          pl.BlockSpec((BR, 8), lambda i: (i, 0)),
      ],
      out_specs=pl.BlockSpec((BR, PW), lambda i: (i, 0)),
      out_shape=jax.ShapeDtypeStruct((N_NODES, PW), jnp.float32),
  )(part2, z, inv)


def _impl(x, edge_index, W1l, b1, W1r, W2l, b2, W2r):
  ei = edge_index.astype(jnp.int32)
  pad1 = NCH1 * CHUNK - EPT1
  dmy1 = jnp.broadcast_to(N_NODES + jnp.arange(pad1, dtype=jnp.int32) % NPAD,
                          (NS, pad1))
  src1 = jnp.concatenate(
      [ei[0].reshape(NS, EPT1), jnp.zeros((NS, pad1), jnp.int32)],
      axis=1).reshape(NS, NCH1, CHUNK)
  dst1 = jnp.concatenate(
      [ei[1].reshape(NS, EPT1), dmy1], axis=1).reshape(NS, NCH1, CHUNK)
  z64 = jnp.zeros((RPT, DH), jnp.float32)
  z16 = jnp.zeros((RPT, PW), jnp.float32)
  w1la = W1l[:DH]
  w1lb = W1l[DH:]
  b1r = b1.reshape(1, D_IN)
  b2r = b2.reshape(1, 3)

  xs = jnp.stack([x[:, :DH], x[:, DH:]])
  part1, cntp = _sc_pass1(xs, src1, dst1, z64, z16)
  p, zz, inv = _tc_mid(part1, cntp, x, w1la, w1lb, b1r, W1r, W2l, W2r, b2r)
  (part2,) = _sc_pass2(p, src1, dst1, z16)
  out16 = _tc_final(part2, zz, inv)
  return out16[:, :3]


kernel = jax.jit(_impl)


# speedup vs baseline: 1.2738x; 1.0430x over previous
import functools

import jax
import jax.numpy as jnp
from jax import lax
from jax.experimental import pallas as pl
from jax.experimental.pallas import tpu as pltpu
from jax.experimental.pallas import tpu_sc as plsc

N_NODES = 10000
N_EDGES = 320000
D_IN = 128
DH = 64
PW = 16
CW = 8

NC = 2
NS = 16
CHUNK = 128
NPAD = 16
NROW = N_NODES + NPAD
RPT = N_NODES // NS
EPT1 = N_EDGES // NS
NCH1 = 160
EPT2 = N_EDGES // (NC * NS)
NCH2 = 80

_SC_PARAMS = pltpu.CompilerParams(use_tc_tiling_on_sc=False)


def _sc_pass1(xs, src_r, dst_r, zrow, zc):
  mesh = plsc.VectorSubcoreMesh(core_axis_name="c", subcore_axis_name="s")
  out_type = [
      jax.ShapeDtypeStruct((NC, N_NODES, DH), jnp.float32),
      jax.ShapeDtypeStruct((NC, N_NODES, CW), jnp.float32),
  ]
  scratch = [
      pltpu.VMEM_SHARED((NROW, DH), jnp.float32),
      pltpu.VMEM_SHARED((NROW, CW), jnp.float32),
      pltpu.VMEM((NCH1, CHUNK), jnp.int32),
      pltpu.VMEM((NCH1, CHUNK), jnp.int32),
      pltpu.VMEM((5, CHUNK, DH), jnp.float32),
      pltpu.VMEM((CHUNK, CW), jnp.float32),
      pltpu.SemaphoreType.DMA((5,)),
      pltpu.SemaphoreType.DMA((5,)),
      pltpu.SemaphoreType.DMA,
  ]

  @functools.partial(pl.kernel, out_type=out_type, mesh=mesh,
                     scratch_types=scratch, compiler_params=_SC_PARAMS)
  def body(x_h, src_h, dst_h, zrow_h, zc_h, out_h, outc_h,
           acc, accc, src_v, dst_v, rows, ones_v, gsem, ssem, csem):
    c = lax.axis_index("c")
    s = lax.axis_index("s")

    pltpu.sync_copy(src_h.at[s], src_v)
    pltpu.sync_copy(dst_h.at[s], dst_v)
    pltpu.sync_copy(zrow_h, acc.at[pl.ds(s * RPT, RPT)])
    pltpu.sync_copy(zc_h, accc.at[pl.ds(s * RPT, RPT)])
    for j in range(CHUNK):
      ones_v[j, :] = jnp.ones((CW,), jnp.float32)
    plsc.subcore_barrier()

    table = x_h.at[c]
    for b in range(4):
      pltpu.async_copy(table.at[src_v.at[b]], rows.at[b], gsem.at[b])

    def group(g, carry):
      for b in range(5):
        i = g * 5 + b
        b4 = (b + 4) % 5
        pltpu.make_async_copy(table.at[src_v.at[i]], rows.at[b],
                              gsem.at[b]).wait()
        pltpu.async_copy(rows.at[b], acc.at[dst_v.at[i]], ssem.at[b],
                         add=True)

        @pl.when(i // (NCH1 // NC) == c)
        def _():
          pltpu.async_copy(ones_v, accc.at[dst_v.at[i]], csem, add=True)

        @pl.when(i >= 1)
        def _():
          pltpu.make_async_copy(rows.at[b4], acc.at[dst_v.at[0]],
                                ssem.at[b4]).wait()

        @pl.when(i + 4 < NCH1)
        def _():
          pltpu.async_copy(table.at[src_v.at[i + 4]], rows.at[b4],
                           gsem.at[b4])

      return carry

    lax.fori_loop(0, NCH1 // 5, group, 0)
    for b in (4,):
      pltpu.make_async_copy(rows.at[b], acc.at[dst_v.at[0]],
                            ssem.at[b]).wait()

    def cdrain(k, carry):
      pltpu.make_async_copy(ones_v, accc.at[dst_v.at[0]], csem).wait()
      return carry

    lax.fori_loop(0, NCH1 // NC, cdrain, 0)
    plsc.subcore_barrier()

    pltpu.sync_copy(acc.at[pl.ds(s * RPT, RPT)],
                    out_h.at[c, pl.ds(s * RPT, RPT)])
    pltpu.sync_copy(accc.at[pl.ds(s * RPT, RPT)],
                    outc_h.at[c, pl.ds(s * RPT, RPT)])

  return body(xs, src_r, dst_r, zrow, zc)


def _sc_pass2(p, src_r, dst_r, z16):
  mesh = plsc.VectorSubcoreMesh(core_axis_name="c", subcore_axis_name="s")
  out_type = [jax.ShapeDtypeStruct((NC, N_NODES, PW), jnp.float32)]
  scratch = [
      pltpu.VMEM_SHARED((NROW, PW), jnp.float32),
      pltpu.VMEM((NCH2, CHUNK), jnp.int32),
      pltpu.VMEM((NCH2, CHUNK), jnp.int32),
      pltpu.VMEM((5, CHUNK, PW), jnp.float32),
      pltpu.SemaphoreType.DMA((5,)),
      pltpu.SemaphoreType.DMA((5,)),
  ]

  @functools.partial(pl.kernel, out_type=out_type, mesh=mesh,
                     scratch_types=scratch, compiler_params=_SC_PARAMS)
  def body(p_h, src_h, dst_h, z16_h, out_h, acc, src_v, dst_v, rows,
           gsem, ssem):
    c = lax.axis_index("c")
    s = lax.axis_index("s")

    pltpu.sync_copy(src_h.at[s, pl.ds(c * NCH2, NCH2)], src_v)
    pltpu.sync_copy(dst_h.at[s, pl.ds(c * NCH2, NCH2)], dst_v)
    pltpu.sync_copy(z16_h, acc.at[pl.ds(s * RPT, RPT)])
    plsc.subcore_barrier()

    for b in range(4):
      pltpu.async_copy(p_h.at[src_v.at[b]], rows.at[b], gsem.at[b])

    def group(g, carry):
      for b in range(5):
        i = g * 5 + b
        b4 = (b + 4) % 5
        pltpu.make_async_copy(p_h.at[src_v.at[i]], rows.at[b],
                              gsem.at[b]).wait()
        pltpu.async_copy(rows.at[b], acc.at[dst_v.at[i]], ssem.at[b],
                         add=True)

        @pl.when(i >= 1)
        def _():
          pltpu.make_async_copy(rows.at[b4], acc.at[dst_v.at[0]],
                                ssem.at[b4]).wait()

        @pl.when(i + 4 < NCH2)
        def _():
          pltpu.async_copy(p_h.at[src_v.at[i + 4]], rows.at[b4],
                           gsem.at[b4])

      return carry

    lax.fori_loop(0, NCH2 // 5, group, 0)
    for b in (4,):
      pltpu.make_async_copy(rows.at[b], acc.at[dst_v.at[0]],
                            ssem.at[b]).wait()
    plsc.subcore_barrier()

    pltpu.sync_copy(acc.at[pl.ds(s * RPT, RPT)],
                    out_h.at[c, pl.ds(s * RPT, RPT)])

  return body(p, src_r, dst_r, z16)


def _tc_mid(part1, cntp, x, w1la, w1lb, b1r, W1r, w2lp, w2rp, b2p):
  BR = 1000
  G = N_NODES // BR

  def body(p1_ref, cp_ref, x_ref, w1la_ref, w1lb_ref, b1_ref, w1r_ref,
           w2l_ref, w2r_ref, b2_ref, p_ref, z_ref, inv_ref):
    cnt8 = cp_ref[0] + cp_ref[1]
    inv8 = 1.0 / jnp.maximum(cnt8, 1.0)
    inv = inv8[:, 0:1]
    h = ((p1_ref[0] * inv) @ w1la_ref[...]
         + (p1_ref[1] * inv) @ w1lb_ref[...]
         + x_ref[...] @ w1r_ref[...] + b1_ref[...])
    h = jnp.maximum(h, 0.0)
    zpad = jnp.zeros((BR, PW - 3), jnp.float32)
    p_ref[...] = jnp.concatenate([h @ w2l_ref[...], zpad], axis=1)
    z_ref[...] = jnp.concatenate([h @ w2r_ref[...] + b2_ref[...], zpad],
                                 axis=1)
    inv_ref[...] = inv8

  return pl.pallas_call(
      body,
      grid=(G,),
      in_specs=[
          pl.BlockSpec((NC, BR, DH), lambda i: (0, i, 0)),
          pl.BlockSpec((NC, BR, CW), lambda i: (0, i, 0)),
          pl.BlockSpec((BR, D_IN), lambda i: (i, 0)),
          pl.BlockSpec((DH, D_IN), lambda i: (0, 0)),
          pl.BlockSpec((DH, D_IN), lambda i: (0, 0)),
          pl.BlockSpec((1, D_IN), lambda i: (0, 0)),
          pl.BlockSpec((D_IN, D_IN), lambda i: (0, 0)),
          pl.BlockSpec((D_IN, 3), lambda i: (0, 0)),
          pl.BlockSpec((D_IN, 3), lambda i: (0, 0)),
          pl.BlockSpec((1, 3), lambda i: (0, 0)),
      ],
      out_specs=[
          pl.BlockSpec((BR, PW), lambda i: (i, 0)),
          pl.BlockSpec((BR, PW), lambda i: (i, 0)),
          pl.BlockSpec((BR, 8), lambda i: (i, 0)),
      ],
      out_shape=[
          jax.ShapeDtypeStruct((N_NODES, PW), jnp.float32),
          jax.ShapeDtypeStruct((N_NODES, PW), jnp.float32),
          jax.ShapeDtypeStruct((N_NODES, 8), jnp.float32),
      ],
  )(part1, cntp, x, w1la, w1lb, b1r, W1r, w2lp, w2rp, b2p)


def _tc_final(part2, z, inv):
  BR = 1000
  G = N_NODES // BR

  def body(p2_ref, z_ref, inv_ref, o_ref):
    agg = p2_ref[0] + p2_ref[1]
    o_ref[...] = agg * inv_ref[:, 0:1] + z_ref[...]

  return pl.pallas_call(
      body,
      grid=(G,),
      in_specs=[
          pl.BlockSpec((NC, BR, PW), lambda i: (0, i, 0)),
          pl.BlockSpec((BR, PW), lambda i: (i, 0)),
          pl.BlockSpec((BR, 8), lambda i: (i, 0)),
      ],
      out_specs=pl.BlockSpec((BR, PW), lambda i: (i, 0)),
      out_shape=jax.ShapeDtypeStruct((N_NODES, PW), jnp.float32),
  )(part2, z, inv)


def _impl(x, edge_index, W1l, b1, W1r, W2l, b2, W2r):
  ei = edge_index.astype(jnp.int32)
  pad1 = NCH1 * CHUNK - EPT1
  dmy1 = jnp.broadcast_to(N_NODES + jnp.arange(pad1, dtype=jnp.int32) % NPAD,
                          (NS, pad1))
  src1 = jnp.concatenate(
      [ei[0].reshape(NS, EPT1), jnp.zeros((NS, pad1), jnp.int32)],
      axis=1).reshape(NS, NCH1, CHUNK)
  dst1 = jnp.concatenate(
      [ei[1].reshape(NS, EPT1), dmy1], axis=1).reshape(NS, NCH1, CHUNK)
  z64 = jnp.zeros((RPT, DH), jnp.float32)
  z16 = jnp.zeros((RPT, PW), jnp.float32)
  zc = jnp.zeros((RPT, CW), jnp.float32)
  w1la = W1l[:DH]
  w1lb = W1l[DH:]
  b1r = b1.reshape(1, D_IN)
  b2r = b2.reshape(1, 3)

  xs = jnp.stack([x[:, :DH], x[:, DH:]])
  part1, cntp = _sc_pass1(xs, src1, dst1, z64, zc)
  p, zz, inv = _tc_mid(part1, cntp, x, w1la, w1lb, b1r, W1r, W2l, W2r, b2r)
  (part2,) = _sc_pass2(p, src1, dst1, z16)
  out16 = _tc_final(part2, zz, inv)
  return out16[:, :3]


kernel = jax.jit(_impl)


# speedup vs baseline: 1.7828x; 1.3995x over previous
import functools

import jax
import jax.numpy as jnp
from jax import lax
from jax.experimental import pallas as pl
from jax.experimental.pallas import tpu as pltpu
from jax.experimental.pallas import tpu_sc as plsc

N_NODES = 10000
N_EDGES = 320000
D_IN = 128
DH = 64
PW = 16
CW = 8

NC = 2
NS = 16
CHUNK = 96
NPAD = 16
NROW = N_NODES + NPAD
RPT = N_NODES // NS
EPT1 = N_EDGES // NS
NCH1 = 210
EPT2 = N_EDGES // (NC * NS)
NCH2 = 105

_SC_PARAMS = pltpu.CompilerParams(use_tc_tiling_on_sc=False)


def _sc_pass1(xs, src_r, dst_r, zrow, zc):
  mesh = plsc.VectorSubcoreMesh(core_axis_name="c", subcore_axis_name="s")
  out_type = [
      jax.ShapeDtypeStruct((NC, N_NODES, DH), jnp.float32),
      jax.ShapeDtypeStruct((NC, N_NODES, CW), jnp.float32),
  ]
  scratch = [
      pltpu.VMEM_SHARED((NROW, DH), jnp.float32),
      pltpu.VMEM_SHARED((NROW, CW), jnp.float32),
      pltpu.VMEM((NCH1, CHUNK), jnp.int32),
      pltpu.VMEM((NCH1, CHUNK), jnp.int32),
      pltpu.VMEM((6, CHUNK, DH), jnp.float32),
      pltpu.VMEM((CHUNK, CW), jnp.float32),
      pltpu.SemaphoreType.DMA((6,)),
      pltpu.SemaphoreType.DMA((6,)),
      pltpu.SemaphoreType.DMA,
  ]

  @functools.partial(pl.kernel, out_type=out_type, mesh=mesh,
                     scratch_types=scratch, compiler_params=_SC_PARAMS)
  def body(x_h, src_h, dst_h, zrow_h, zc_h, out_h, outc_h,
           acc, accc, src_v, dst_v, rows, ones_v, gsem, ssem, csem):
    c = lax.axis_index("c")
    s = lax.axis_index("s")

    pltpu.sync_copy(src_h.at[s], src_v)
    pltpu.sync_copy(dst_h.at[s], dst_v)
    pltpu.sync_copy(zrow_h, acc.at[pl.ds(s * RPT, RPT)])
    pltpu.sync_copy(zc_h, accc.at[pl.ds(s * RPT, RPT)])
    for j in range(CHUNK):
      ones_v[j, :] = jnp.ones((CW,), jnp.float32)
    plsc.subcore_barrier()

    table = x_h.at[c]
    for b in range(5):
      pltpu.async_copy(table.at[src_v.at[b]], rows.at[b], gsem.at[b])

    def group(g, carry):
      for b in range(6):
        i = g * 6 + b
        b5 = (b + 5) % 6
        pltpu.make_async_copy(table.at[src_v.at[i]], rows.at[b],
                              gsem.at[b]).wait()
        pltpu.async_copy(rows.at[b], acc.at[dst_v.at[i]], ssem.at[b],
                         add=True)

        @pl.when(i // (NCH1 // NC) == c)
        def _():
          pltpu.async_copy(ones_v, accc.at[dst_v.at[i]], csem, add=True)

        @pl.when(i >= 1)
        def _():
          pltpu.make_async_copy(rows.at[b5], acc.at[dst_v.at[0]],
                                ssem.at[b5]).wait()

        @pl.when(i + 5 < NCH1)
        def _():
          pltpu.async_copy(table.at[src_v.at[i + 5]], rows.at[b5],
                           gsem.at[b5])

      return carry

    lax.fori_loop(0, NCH1 // 6, group, 0)
    for b in (5,):
      pltpu.make_async_copy(rows.at[b], acc.at[dst_v.at[0]],
                            ssem.at[b]).wait()

    def cdrain(k, carry):
      pltpu.make_async_copy(ones_v, accc.at[dst_v.at[0]], csem).wait()
      return carry

    lax.fori_loop(0, NCH1 // NC, cdrain, 0)
    plsc.subcore_barrier()

    pltpu.sync_copy(acc.at[pl.ds(s * RPT, RPT)],
                    out_h.at[c, pl.ds(s * RPT, RPT)])
    pltpu.sync_copy(accc.at[pl.ds(s * RPT, RPT)],
                    outc_h.at[c, pl.ds(s * RPT, RPT)])

  return body(xs, src_r, dst_r, zrow, zc)


def _sc_pass2(p, src_r, dst_r, z16):
  mesh = plsc.VectorSubcoreMesh(core_axis_name="c", subcore_axis_name="s")
  out_type = [jax.ShapeDtypeStruct((NC, N_NODES, PW), jnp.float32)]
  scratch = [
      pltpu.VMEM_SHARED((NROW, PW), jnp.float32),
      pltpu.VMEM((NCH2, CHUNK), jnp.int32),
      pltpu.VMEM((NCH2, CHUNK), jnp.int32),
      pltpu.VMEM((5, CHUNK, PW), jnp.float32),
      pltpu.SemaphoreType.DMA((5,)),
      pltpu.SemaphoreType.DMA((5,)),
  ]

  @functools.partial(pl.kernel, out_type=out_type, mesh=mesh,
                     scratch_types=scratch, compiler_params=_SC_PARAMS)
  def body(p_h, src_h, dst_h, z16_h, out_h, acc, src_v, dst_v, rows,
           gsem, ssem):
    c = lax.axis_index("c")
    s = lax.axis_index("s")

    pltpu.sync_copy(src_h.at[s, pl.ds(c * NCH2, NCH2)], src_v)
    pltpu.sync_copy(dst_h.at[s, pl.ds(c * NCH2, NCH2)], dst_v)
    pltpu.sync_copy(z16_h, acc.at[pl.ds(s * RPT, RPT)])
    plsc.subcore_barrier()

    for b in range(4):
      pltpu.async_copy(p_h.at[src_v.at[b]], rows.at[b], gsem.at[b])

    def group(g, carry):
      for b in range(5):
        i = g * 5 + b
        b4 = (b + 4) % 5
        pltpu.make_async_copy(p_h.at[src_v.at[i]], rows.at[b],
                              gsem.at[b]).wait()
        pltpu.async_copy(rows.at[b], acc.at[dst_v.at[i]], ssem.at[b],
                         add=True)

        @pl.when(i >= 1)
        def _():
          pltpu.make_async_copy(rows.at[b4], acc.at[dst_v.at[0]],
                                ssem.at[b4]).wait()

        @pl.when(i + 4 < NCH2)
        def _():
          pltpu.async_copy(p_h.at[src_v.at[i + 4]], rows.at[b4],
                           gsem.at[b4])

      return carry

    lax.fori_loop(0, NCH2 // 5, group, 0)
    for b in (4,):
      pltpu.make_async_copy(rows.at[b], acc.at[dst_v.at[0]],
                            ssem.at[b]).wait()
    plsc.subcore_barrier()

    pltpu.sync_copy(acc.at[pl.ds(s * RPT, RPT)],
                    out_h.at[c, pl.ds(s * RPT, RPT)])

  return body(p, src_r, dst_r, z16)


def _tc_mid(part1, cntp, x, w1la, w1lb, b1r, W1r, w2lp, w2rp, b2p):
  BR = 1000
  G = N_NODES // BR

  def body(p1_ref, cp_ref, x_ref, w1la_ref, w1lb_ref, b1_ref, w1r_ref,
           w2l_ref, w2r_ref, b2_ref, p_ref, z_ref, inv_ref):
    cnt8 = cp_ref[0] + cp_ref[1]
    inv8 = 1.0 / jnp.maximum(cnt8, 1.0)
    inv = inv8[:, 0:1]
    h = ((p1_ref[0] * inv) @ w1la_ref[...]
         + (p1_ref[1] * inv) @ w1lb_ref[...]
         + x_ref[...] @ w1r_ref[...] + b1_ref[...])
    h = jnp.maximum(h, 0.0)
    zpad = jnp.zeros((BR, PW - 3), jnp.float32)
    p_ref[...] = jnp.concatenate([h @ w2l_ref[...], zpad], axis=1)
    z_ref[...] = jnp.concatenate([h @ w2r_ref[...] + b2_ref[...], zpad],
                                 axis=1)
    inv_ref[...] = inv8

  return pl.pallas_call(
      body,
      grid=(G,),
      in_specs=[
          pl.BlockSpec((NC, BR, DH), lambda i: (0, i, 0)),
          pl.BlockSpec((NC, BR, CW), lambda i: (0, i, 0)),
          pl.BlockSpec((BR, D_IN), lambda i: (i, 0)),
          pl.BlockSpec((DH, D_IN), lambda i: (0, 0)),
          pl.BlockSpec((DH, D_IN), lambda i: (0, 0)),
          pl.BlockSpec((1, D_IN), lambda i: (0, 0)),
          pl.BlockSpec((D_IN, D_IN), lambda i: (0, 0)),
          pl.BlockSpec((D_IN, 3), lambda i: (0, 0)),
          pl.BlockSpec((D_IN, 3), lambda i: (0, 0)),
          pl.BlockSpec((1, 3), lambda i: (0, 0)),
      ],
      out_specs=[
          pl.BlockSpec((BR, PW), lambda i: (i, 0)),
          pl.BlockSpec((BR, PW), lambda i: (i, 0)),
          pl.BlockSpec((BR, 8), lambda i: (i, 0)),
      ],
      out_shape=[
          jax.ShapeDtypeStruct((N_NODES, PW), jnp.float32),
          jax.ShapeDtypeStruct((N_NODES, PW), jnp.float32),
          jax.ShapeDtypeStruct((N_NODES, 8), jnp.float32),
      ],
  )(part1, cntp, x, w1la, w1lb, b1r, W1r, w2lp, w2rp, b2p)


def _tc_final(part2, z, inv):
  BR = 1000
  G = N_NODES // BR

  def body(p2_ref, z_ref, inv_ref, o_ref):
    agg = p2_ref[0] + p2_ref[1]
    o_ref[...] = agg * inv_ref[:, 0:1] + z_ref[...]

  return pl.pallas_call(
      body,
      grid=(G,),
      in_specs=[
          pl.BlockSpec((NC, BR, PW), lambda i: (0, i, 0)),
          pl.BlockSpec((BR, PW), lambda i: (i, 0)),
          pl.BlockSpec((BR, 8), lambda i: (i, 0)),
      ],
      out_specs=pl.BlockSpec((BR, PW), lambda i: (i, 0)),
      out_shape=jax.ShapeDtypeStruct((N_NODES, PW), jnp.float32),
  )(part2, z, inv)


def _impl(x, edge_index, W1l, b1, W1r, W2l, b2, W2r):
  ei = edge_index.astype(jnp.int32)
  pad1 = NCH1 * CHUNK - EPT1
  dmy1 = jnp.broadcast_to(N_NODES + jnp.arange(pad1, dtype=jnp.int32) % NPAD,
                          (NS, pad1))
  src1 = jnp.concatenate(
      [ei[0].reshape(NS, EPT1), jnp.zeros((NS, pad1), jnp.int32)],
      axis=1).reshape(NS, NCH1, CHUNK)
  dst1 = jnp.concatenate(
      [ei[1].reshape(NS, EPT1), dmy1], axis=1).reshape(NS, NCH1, CHUNK)
  z64 = jnp.zeros((RPT, DH), jnp.float32)
  z16 = jnp.zeros((RPT, PW), jnp.float32)
  zc = jnp.zeros((RPT, CW), jnp.float32)
  w1la = W1l[:DH]
  w1lb = W1l[DH:]
  b1r = b1.reshape(1, D_IN)
  b2r = b2.reshape(1, 3)

  xs = jnp.stack([x[:, :DH], x[:, DH:]])
  part1, cntp = _sc_pass1(xs, src1, dst1, z64, zc)
  p, zz, inv = _tc_mid(part1, cntp, x, w1la, w1lb, b1r, W1r, W2l, W2r, b2r)
  (part2,) = _sc_pass2(p, src1, dst1, z16)
  out16 = _tc_final(part2, zz, inv)
  return out16[:, :3]


kernel = jax.jit(_impl)
